# baseline (speedup 1.0000x reference)
import functools

import jax
import jax.numpy as jnp
from jax.experimental import pallas as pl
from jax.experimental.pallas import tpu as pltpu


def _window_dot(xw, w_bf, wo, stride, cin, ksz):
    lo = wo * stride * cin
    slab = xw[:, lo:lo + ksz * cin]
    return jnp.dot(slab, w_bf, preferred_element_type=jnp.float32)


def _relayout_stats_kernel(x_ref, w_ref, o_ref, psum_ref, psq_ref,
                           acc_s, acc_q, *, stride, wout, ksz, n_tiles):
    t = pl.program_id(0)
    nb, cin, h, w = x_ref.shape
    cout = w_ref.shape[1]
    rows = nb * h

    @pl.when(t == 0)
    def _init():
        acc_s[...] = jnp.zeros_like(acc_s)
        acc_q[...] = jnp.zeros_like(acc_q)

    xw = (jnp.transpose(x_ref[...].astype(jnp.bfloat16), (0, 2, 3, 1))
          .reshape(rows, w * cin))
    o_ref[...] = xw

    w_bf = w_ref[...].astype(jnp.bfloat16)
    ty = jnp.zeros((rows, cout), jnp.float32)
    tq = jnp.zeros((rows, cout), jnp.float32)
    for wo in range(wout):
        y = _window_dot(xw, w_bf, wo, stride, cin, ksz)
        ty = ty + y
        tq = tq + y * y
    acc_s[...] += jnp.sum(ty, axis=0, keepdims=True)
    acc_q[...] += jnp.sum(tq, axis=0, keepdims=True)

    @pl.when(t == n_tiles - 1)
    def _flush():
        psum_ref[...] = acc_s[...]
        psq_ref[...] = acc_q[...]


def _norm_kernel(xw_ref, w_ref, g_ref, b_ref, psum_ref, psq_ref, o_ref,
                 *, stride, wout, ksz, cin, inv_m, eps):
    tile_r = xw_ref.shape[0]
    cout = w_ref.shape[1]
    n_blk = o_ref.shape[0]
    h = o_ref.shape[2]

    mean = psum_ref[...] * inv_m
    var = psq_ref[...] * inv_m - mean * mean
    scale = g_ref[...] * jax.lax.rsqrt(var + eps)
    shift = b_ref[...] - mean * scale
    w_bf = (w_ref[...] * scale).astype(jnp.bfloat16)
    shift_b = jnp.broadcast_to(shift, (tile_r, cout))
    for wo in range(wout):
        y = _window_dot(xw_ref[...], w_bf, wo, stride, cin, ksz)
        y = jnp.maximum(y + shift_b, 0.0)
        o_ref[:, wo, :, :] = y.reshape(n_blk, h, cout)


def kernel(x, conv_w, conv_b, bn_gamma, bn_beta):
    del conv_b
    N, Cin, H, W = x.shape
    Cout = conv_w.shape[0]
    K = conv_w.shape[3]
    S = 2
    Wout = (W - K) // S + 1
    R = N * H
    eps = 1e-5

    NB0 = min(2, N)
    TILE_R = min(256, R)
    N_BLK = TILE_R // H
    assert R % TILE_R == 0 and TILE_R % H == 0 and N % N_BLK == 0
    assert N % NB0 == 0
    n_tiles = R // TILE_R

    w_mat = conv_w.reshape(Cout, Cin, K).transpose(2, 1, 0).reshape(K * Cin, Cout)
    w_mat = w_mat.astype(jnp.float32)

    g_row = bn_gamma.reshape(1, Cout).astype(jnp.float32)
    b_row = bn_beta.reshape(1, Cout).astype(jnp.float32)
    inv_m = 1.0 / float(N * H * Wout)

    xw, psum, psq = pl.pallas_call(
        functools.partial(_relayout_stats_kernel, stride=S, wout=Wout,
                          ksz=K, n_tiles=N // NB0),
        out_shape=[jax.ShapeDtypeStruct((R, W * Cin), jnp.bfloat16),
                   jax.ShapeDtypeStruct((1, Cout), jnp.float32),
                   jax.ShapeDtypeStruct((1, Cout), jnp.float32)],
        grid=(N // NB0,),
        in_specs=[pl.BlockSpec((NB0, Cin, H, W), lambda i: (i, 0, 0, 0)),
                  pl.BlockSpec((K * Cin, Cout), lambda i: (0, 0))],
        out_specs=[pl.BlockSpec((NB0 * H, W * Cin), lambda i: (i, 0)),
                   pl.BlockSpec((1, Cout), lambda i: (0, 0)),
                   pl.BlockSpec((1, Cout), lambda i: (0, 0))],
        scratch_shapes=[pltpu.VMEM((1, Cout), jnp.float32),
                        pltpu.VMEM((1, Cout), jnp.float32)],
        compiler_params=pltpu.CompilerParams(
            dimension_semantics=("arbitrary",),
            vmem_limit_bytes=48 * 1024 * 1024,
        ),
        cost_estimate=pl.CostEstimate(
            flops=2 * R * Wout * Cout * Cin * K,
            transcendentals=0,
            bytes_accessed=int(x.size * 4 + x.size * 2),
        ),
    )(x, w_mat)

    out_nwhc = pl.pallas_call(
        functools.partial(_norm_kernel, stride=S, wout=Wout, ksz=K, cin=Cin,
                          inv_m=inv_m, eps=eps),
        out_shape=jax.ShapeDtypeStruct((N, Wout, H, Cout), jnp.float32),
        grid=(n_tiles,),
        in_specs=[
            pl.BlockSpec((TILE_R, W * Cin), lambda i: (i, 0)),
            pl.BlockSpec((K * Cin, Cout), lambda i: (0, 0)),
            pl.BlockSpec((1, Cout), lambda i: (0, 0)),
            pl.BlockSpec((1, Cout), lambda i: (0, 0)),
            pl.BlockSpec((1, Cout), lambda i: (0, 0)),
            pl.BlockSpec((1, Cout), lambda i: (0, 0)),
        ],
        out_specs=pl.BlockSpec((N_BLK, Wout, H, Cout),
                               lambda i: (i, 0, 0, 0)),
        compiler_params=pltpu.CompilerParams(
            dimension_semantics=("arbitrary",),
            vmem_limit_bytes=48 * 1024 * 1024,
        ),
        cost_estimate=pl.CostEstimate(
            flops=2 * R * Wout * Cout * Cin * K,
            transcendentals=Cout,
            bytes_accessed=int(xw.size * 2 + w_mat.size * 4
                               + N * Wout * H * Cout * 4),
        ),
    )(xw, w_mat, g_row, b_row, psum, psq)

    return jnp.transpose(out_nwhc, (0, 3, 2, 1))


# speedup vs baseline: 1.0692x; 1.0692x over previous
import functools

import jax
import jax.numpy as jnp
from jax.experimental import pallas as pl
from jax.experimental.pallas import tpu as pltpu


def _round_up(a, b):
    return (a + b - 1) // b * b


def _relayout_kernel(x_ref, o_ref):
    nb, cin, h, w = x_ref.shape
    o_ref[...] = (jnp.transpose(x_ref[...].astype(jnp.bfloat16), (0, 2, 3, 1))
                  .reshape(nb * h, w * cin))


def _fused_kernel(xw_ref, w_ref, g_ref, b_ref, o_ref, acc_s, acc_q,
                  *, stride, wout, ksz, cin, inv_m, eps):
    sweep = pl.program_id(0)
    t = pl.program_id(1)
    tile_r = xw_ref.shape[0]
    cout = w_ref.shape[1]
    lanes_per_w = stride * cin

    @pl.when(jnp.logical_and(sweep == 0, t == 0))
    def _init():
        acc_s[...] = jnp.zeros_like(acc_s)
        acc_q[...] = jnp.zeros_like(acc_q)

    def conv_t(wo, w_bf):
        slab = xw_ref[:, pl.ds(wo * lanes_per_w, ksz * cin)]
        return jnp.dot(slab, w_bf, preferred_element_type=jnp.float32)

    @pl.when(sweep == 0)
    def _stats():
        w_bf = w_ref[...].astype(jnp.bfloat16)
        ty = jnp.zeros((tile_r, cout), jnp.float32)
        tq = jnp.zeros((tile_r, cout), jnp.float32)
        for wo in range(wout):
            y = conv_t(wo, w_bf)
            ty = ty + y
            tq = tq + y * y
        acc_s[...] += jnp.sum(ty, axis=0, keepdims=True)
        acc_q[...] += jnp.sum(tq, axis=0, keepdims=True)

    @pl.when(sweep == 1)
    def _normalize():
        n_blk = o_ref.shape[0]
        h = o_ref.shape[2]
        mean = acc_s[...] * inv_m
        var = acc_q[...] * inv_m - mean * mean
        scale = g_ref[...] * jax.lax.rsqrt(var + eps)
        shift = b_ref[...] - mean * scale
        w_bf = (w_ref[...] * scale).astype(jnp.bfloat16)
        shift_b = jnp.broadcast_to(shift, (tile_r, cout))
        for wo in range(wout):
            y = jnp.maximum(conv_t(wo, w_bf) + shift_b, 0.0)
            o_ref[:, wo, :, :] = y.reshape(n_blk, h, cout)


def kernel(x, conv_w, conv_b, bn_gamma, bn_beta):
    del conv_b
    N, Cin, H, W = x.shape
    Cout = conv_w.shape[0]
    K = conv_w.shape[3]
    S = 2
    Wout = (W - K) // S + 1
    R = N * H
    eps = 1e-5

    TILE_R = min(256, R)
    N_BLK = TILE_R // H
    assert R % TILE_R == 0 and TILE_R % H == 0 and N % N_BLK == 0
    n_tiles = R // TILE_R

    NB0 = min(2, N)
    xw = pl.pallas_call(
        _relayout_kernel,
        out_shape=jax.ShapeDtypeStruct((R, W * Cin), jnp.bfloat16),
        grid=(N // NB0,),
        in_specs=[pl.BlockSpec((NB0, Cin, H, W), lambda i: (i, 0, 0, 0))],
        out_specs=pl.BlockSpec((NB0 * H, W * Cin), lambda i: (i, 0)),
        compiler_params=pltpu.CompilerParams(
            dimension_semantics=("arbitrary",),
            vmem_limit_bytes=48 * 1024 * 1024,
        ),
        cost_estimate=pl.CostEstimate(
            flops=0, transcendentals=0,
            bytes_accessed=int(x.size * 4 + x.size * 2),
        ),
    )(x)

    w_mat = conv_w.reshape(Cout, Cin, K).transpose(2, 1, 0).reshape(K * Cin, Cout)
    w_mat = w_mat.astype(jnp.float32)

    g_row = bn_gamma.reshape(1, Cout).astype(jnp.float32)
    b_row = bn_beta.reshape(1, Cout).astype(jnp.float32)
    inv_m = 1.0 / float(N * H * Wout)

    out_nwhc = pl.pallas_call(
        functools.partial(_fused_kernel, stride=S, wout=Wout, ksz=K, cin=Cin,
                          inv_m=inv_m, eps=eps),
        out_shape=jax.ShapeDtypeStruct((N, Wout, H, Cout), jnp.float32),
        grid=(2, n_tiles),
        in_specs=[
            pl.BlockSpec((TILE_R, W * Cin), lambda s, i: (i, 0)),
            pl.BlockSpec((K * Cin, Cout), lambda s, i: (0, 0)),
            pl.BlockSpec((1, Cout), lambda s, i: (0, 0)),
            pl.BlockSpec((1, Cout), lambda s, i: (0, 0)),
        ],
        out_specs=pl.BlockSpec((N_BLK, Wout, H, Cout),
                               lambda s, i: (s * i, 0, 0, 0)),
        scratch_shapes=[pltpu.VMEM((1, Cout), jnp.float32),
                        pltpu.VMEM((1, Cout), jnp.float32)],
        compiler_params=pltpu.CompilerParams(
            dimension_semantics=("arbitrary", "arbitrary"),
            vmem_limit_bytes=48 * 1024 * 1024,
        ),
        cost_estimate=pl.CostEstimate(
            flops=2 * 2 * R * Wout * Cout * Cin * K,
            transcendentals=Cout * n_tiles,
            bytes_accessed=int(2 * xw.size * 2 + w_mat.size * 4
                               + N * Wout * H * Cout * 4),
        ),
    )(xw, w_mat, g_row, b_row)

    return jnp.transpose(out_nwhc, (0, 3, 2, 1))
